# Initial kernel scaffold; baseline (speedup 1.0000x reference)
import jax
import jax.numpy as jnp
from jax.experimental import pallas as pl


def kernel(x, uncertainty, edge_index, edge_lengths, refinement_level, W1, b1, W2, b2, W3, b3):
    raise NotImplementedError("write your pallas kernel here")


# speedup vs baseline: 53.8623x; 53.8623x over previous
import functools

import jax
import jax.numpy as jnp
from jax import lax
from jax.experimental import pallas as pl
from jax.experimental.pallas import tpu as pltpu
from jax.experimental.pallas import tpu_sc as plsc

N_NODES = 100000
NP = 100352
N_EDGES = 3200000
NEP = 3211264
EROWS = NEP // 128
NC, NS = 2, 16
NW = NC * NS
RPT = EROWS // NW
K = 8
CHUNKS = RPT // K
SLICE = NP // NS
NP2 = NP // 2
MAXL = 3


def _pre_body(xt_ref, ut_ref, tbl_ref):
    u = ut_ref[...]
    tbl_ref[0:3, :] = xt_ref[...]
    tbl_ref[3:4, :] = jnp.sqrt(jnp.sum(u * u, axis=0, keepdims=True))


def _pre(xt, ut):
    bc = 2048
    return pl.pallas_call(
        _pre_body,
        grid=(NP // bc,),
        in_specs=[pl.BlockSpec((3, bc), lambda b: (0, b)),
                  pl.BlockSpec((4, bc), lambda b: (0, b))],
        out_specs=pl.BlockSpec((4, bc), lambda b: (0, b)),
        out_shape=jax.ShapeDtypeStruct((4, NP), jnp.float32),
    )(xt, ut)


def _sc_body(t0_hbm, t1_hbm, t2_hbm, t3_hbm, src_hbm, dst_hbm, len_hbm,
             adds_hbm, maxp_hbm, und_hbm,
             src_v, dst_v, len_v, xs_v, xd_v, grad_v, el_v, ones_v,
             maxacc, nbuf, zb, tb0, tb1, tb2, tb3, acc_g, acc_e, acc_d,
             sem):
    cid = lax.axis_index("c")
    sid = lax.axis_index("s")
    wid = cid * NS + sid

    iota = lax.iota(jnp.int32, 16)
    zf = jnp.zeros((16,), jnp.float32)
    onef = jnp.ones((16,), jnp.float32)

    nbuf[pl.ds(16, 16)] = jnp.full((16,), -1, jnp.int32)
    def _zzb(i, carry):
        zb[pl.ds(i * 16, 16)] = zf
        return carry
    lax.fori_loop(0, 64, _zzb, 0)

    def _ones(i, carry):
        k16 = i * 16
        ones_v[k16 >> 7, pl.ds(k16 & 127, 16)] = onef
        return carry
    lax.fori_loop(0, 64, _ones, 0)

    for th, tb in zip((t0_hbm, t1_hbm, t2_hbm, t3_hbm), (tb0, tb1, tb2, tb3)):
        def _stage(c, carry, th=th, tb=tb):
            col0 = sid * SLICE + c * 896
            pltpu.sync_copy(th.at[pl.ds(col0, 896)], zb.at[pl.ds(0, 896)])
            pltpu.sync_copy(zb.at[pl.ds(0, 896)], tb.at[pl.ds(col0, 896)])
            return carry
        lax.fori_loop(0, 7, _stage, 0)

    def _zzb2(i, carry):
        zb[pl.ds(i * 16, 16)] = zf
        return carry
    lax.fori_loop(0, 64, _zzb2, 0)

    for acc in (acc_g, acc_e, acc_d):
        def _zacc(c, carry, acc=acc):
            col0 = sid * SLICE + c * 896
            pltpu.sync_copy(zb.at[pl.ds(0, 896)], acc.at[pl.ds(col0, 896)])
            return carry
        lax.fori_loop(0, 7, _zacc, 0)
    plsc.subcore_barrier()

    base_row = wid * RPT

    def _chunk(cn, carry):
        r0 = base_row + cn * K
        pltpu.sync_copy(src_hbm.at[pl.ds(r0, K)], src_v)
        pltpu.sync_copy(dst_hbm.at[pl.ds(r0, K)], dst_v)
        pltpu.sync_copy(len_hbm.at[pl.ds(r0, K)], len_v)
        descs = []
        for j in range(K):
            for r, tb in enumerate((tb0, tb1, tb2, tb3)):
                descs.append(pltpu.async_copy(
                    tb.at[src_v.at[j]], xs_v.at[r, j], sem))
                descs.append(pltpu.async_copy(
                    tb.at[dst_v.at[j]], xd_v.at[r, j], sem))
        for d in descs:
            d.wait()
        pltpu.sync_copy(xd_v.at[3], und_hbm.at[pl.ds(r0, K)])

        def _grp(g, inner):
            jv = g >> 3
            k0 = (g & 7) * 16
            xs0 = xs_v[0, jv, pl.ds(k0, 16)]
            xs1 = xs_v[1, jv, pl.ds(k0, 16)]
            xs2 = xs_v[2, jv, pl.ds(k0, 16)]
            uns = xs_v[3, jv, pl.ds(k0, 16)]
            xd0 = xd_v[0, jv, pl.ds(k0, 16)]
            xd1 = xd_v[1, jv, pl.ds(k0, 16)]
            xd2 = xd_v[2, jv, pl.ds(k0, 16)]
            und = xd_v[3, jv, pl.ds(k0, 16)]
            dx = xd0 - xs0
            dy = xd1 - xs1
            dz = xd2 - xs2
            d2 = dx * dx + dy * dy + dz * dz
            z = lax.bitcast_convert_type(
                jnp.int32(0x5F3759DF) -
                (lax.bitcast_convert_type(d2, jnp.int32) >> 1), jnp.float32)
            z = z * (1.5 - 0.5 * d2 * z * z)
            z = z * (1.5 - 0.5 * d2 * z * z)
            z = z * (1.5 - 0.5 * d2 * z * z)
            el_v[jv, pl.ds(k0, 16)] = d2 * z
            elen = len_v[jv, pl.ds(k0, 16)]
            grad_v[jv, pl.ds(k0, 16)] = jnp.abs(und - uns) / (elen + 1e-8)
            return inner
        lax.fori_loop(0, 64, _grp, 0)

        for j in range(K):
            pltpu.sync_copy(grad_v.at[j], acc_g.at[src_v.at[j]], add=True)
            pltpu.sync_copy(el_v.at[j], acc_e.at[src_v.at[j]], add=True)
            pltpu.sync_copy(ones_v.at[j], acc_d.at[src_v.at[j]], add=True)
        return carry
    lax.fori_loop(0, CHUNKS, _chunk, 0)

    for h in range(2):
        def _zm(i, carry):
            maxacc[pl.ds(i * 16, 16)] = zf
            return carry
        lax.fori_loop(0, NP2 // 16, _zm, 0)
        lo = h * NP2

        def _mchunk(cn, carry, lo=lo):
            r0 = base_row + cn * K
            pltpu.sync_copy(src_hbm.at[pl.ds(r0, K)], src_v)
            pltpu.sync_copy(und_hbm.at[pl.ds(r0, K)], len_v)

            def _mgrp(g, inner):
                jv = g >> 3
                k0 = (g & 7) * 16
                s16 = src_v[jv, pl.ds(k0, 16)]
                und = len_v[jv, pl.ds(k0, 16)]
                ub = lax.bitcast_convert_type(und, jnp.uint32)
                key = (s16.astype(jnp.uint32) << 15) | (ub >> 16)
                sk, sv = plsc.sort_key_val(key, und)
                sn = lax.shift_right_logical(
                    sk, jnp.uint32(15)).astype(jnp.int32)
                nbuf[pl.ds(0, 16)] = sn
                nxt = plsc.load_gather(nbuf, [iota + 1])
                m = ((sn != nxt) | (iota == 15)) & \
                    (sn >= lo) & (sn < lo + NP2)
                si = jnp.where(m, sn - lo, 0)
                cur = plsc.load_gather(maxacc, [si], mask=m)
                plsc.store_scatter(maxacc, [si], jnp.maximum(cur, sv),
                                   mask=m)
                return inner
            lax.fori_loop(0, 64, _mgrp, 0)
            return carry
        lax.fori_loop(0, CHUNKS, _mchunk, 0)
        pltpu.sync_copy(maxacc, maxp_hbm.at[pl.ds(wid * NP + lo, NP2)])

    plsc.subcore_barrier()

    for a, acc in enumerate((acc_g, acc_e, acc_d)):
        def _out(c, carry, a=a, acc=acc):
            col0 = sid * SLICE + c * 896
            pltpu.sync_copy(acc.at[pl.ds(col0, 896)], zb.at[pl.ds(0, 896)])
            pltpu.sync_copy(zb.at[pl.ds(0, 896)],
                            adds_hbm.at[pl.ds((cid * 3 + a) * NP + col0,
                                              896)])
            return carry
        lax.fori_loop(0, 7, _out, 0)


_sc_mesh = plsc.VectorSubcoreMesh(core_axis_name="c", subcore_axis_name="s",
                                  num_cores=NC, num_subcores=NS)
_sc_call = functools.partial(
    pl.kernel,
    out_type=[jax.ShapeDtypeStruct((NC * 3 * NP,), jnp.float32),
              jax.ShapeDtypeStruct((NW * NP,), jnp.float32),
              jax.ShapeDtypeStruct((EROWS, 128), jnp.float32)],
    mesh=_sc_mesh,
    compiler_params=pltpu.CompilerParams(needs_layout_passes=False),
    scratch_types=[
        pltpu.VMEM((K, 128), jnp.int32),
        pltpu.VMEM((K, 128), jnp.int32),
        pltpu.VMEM((K, 128), jnp.float32),
        pltpu.VMEM((4, K, 128), jnp.float32),
        pltpu.VMEM((4, K, 128), jnp.float32),
        pltpu.VMEM((K, 128), jnp.float32),
        pltpu.VMEM((K, 128), jnp.float32),
        pltpu.VMEM((K, 128), jnp.float32),
        pltpu.VMEM((NP2,), jnp.float32),
        pltpu.VMEM((32,), jnp.int32),
        pltpu.VMEM((1024,), jnp.float32),
        pltpu.VMEM_SHARED((NP,), jnp.float32),
        pltpu.VMEM_SHARED((NP,), jnp.float32),
        pltpu.VMEM_SHARED((NP,), jnp.float32),
        pltpu.VMEM_SHARED((NP,), jnp.float32),
        pltpu.VMEM_SHARED((NP,), jnp.float32),
        pltpu.VMEM_SHARED((NP,), jnp.float32),
        pltpu.VMEM_SHARED((NP,), jnp.float32),
        pltpu.SemaphoreType.DMA,
    ],
)(_sc_body)


_BC = 512


def _post_body(a_ref, mx_ref, tbl_ref, rl_ref, w1_ref, b1_ref, w2_ref,
               b2_ref, w3_ref, b3_ref, probs_ref, dec_ref, cnt_ref,
               feat, cmm):
    p = pl.program_id(0)
    b = pl.program_id(1)
    col0 = b * _BC
    cols = col0 + lax.broadcasted_iota(jnp.int32, (1, _BC), 1)
    real = cols < N_NODES

    @pl.when(p == 0)
    def _phase0():
        a0 = a_ref[0]
        a1 = a_ref[1]
        deg = a0[2:3] + a1[2:3]
        grad = (a0[0:1] + a1[0:1]) / (deg + 1e-8)
        curv = (a0[1:2] + a1[1:2]) / (deg + 1e-8)
        nmax = jnp.max(mx_ref[...], axis=0, keepdims=True)
        feat[0:1, pl.ds(col0, _BC)] = tbl_ref[3:4, :]
        feat[1:2, pl.ds(col0, _BC)] = grad
        feat[2:3, pl.ds(col0, _BC)] = curv
        feat[3:4, pl.ds(col0, _BC)] = nmax

        @pl.when(b == 0)
        def _init():
            cmm[0] = jnp.float32(3.0e38)
            cmm[1] = jnp.float32(-3.0e38)
        bmin = jnp.min(jnp.where(real, curv, jnp.float32(3.0e38)))
        bmax = jnp.max(jnp.where(real, curv, jnp.float32(-3.0e38)))
        cmm[0] = jnp.minimum(cmm[0], bmin)
        cmm[1] = jnp.maximum(cmm[1], bmax)

    @pl.when(p == 1)
    def _phase1():
        f = feat[:, pl.ds(col0, _BC)]
        curv = (f[2:3] - cmm[0]) / (cmm[1] - cmm[0] + 1e-8)
        feats = jnp.concatenate([f[0:2], curv, f[3:4]], axis=0)
        h = jnp.dot(w1_ref[...], feats,
                    preferred_element_type=jnp.float32) + b1_ref[...]
        h = jnp.maximum(h, 0.0)
        h = jnp.dot(w2_ref[...], h,
                    preferred_element_type=jnp.float32) + b2_ref[...]
        h = jnp.maximum(h, 0.0)
        lg = jnp.dot(w3_ref[...], h,
                     preferred_element_type=jnp.float32) + b3_ref[...]
        m = jnp.max(lg, axis=0, keepdims=True)
        e = jnp.exp(lg - m)
        probs_ref[...] = e / jnp.sum(e, axis=0, keepdims=True)
        l0 = lg[0:1]
        l1 = lg[1:2]
        l2 = lg[2:3]
        d = jnp.where(l1 > l0, 1, 0)
        d = jnp.where(l2 > jnp.maximum(l0, l1), 2, d)
        rl = rl_ref[...]
        d = jnp.where((rl >= MAXL) & (d == 0), 1, d)
        d = jnp.where((rl <= 0) & (d == 2), 1, d)
        d = d.astype(jnp.int32)
        dec_ref[...] = d

        @pl.when(b == 0)
        def _init():
            cnt_ref[0, 0] = 0
            cnt_ref[0, 1] = 0
            cnt_ref[0, 2] = 0
        cnt_ref[0, 0] += jnp.sum(jnp.where((d == 0) & real, 1, 0))
        cnt_ref[0, 1] += jnp.sum(jnp.where((d == 1) & real, 1, 0))
        cnt_ref[0, 2] += jnp.sum(jnp.where((d == 2) & real, 1, 0))


def _post(addsT, mx, tblT, rl2d, w1t, b1c, w2t, b2c, w3t, b3c):
    nb = NP // _BC
    return pl.pallas_call(
        _post_body,
        grid=(2, nb),
        in_specs=[
            pl.BlockSpec((2, 3, _BC), lambda p, b: (0, 0, b)),
            pl.BlockSpec((NW, _BC), lambda p, b: (0, b)),
            pl.BlockSpec((4, _BC), lambda p, b: (0, b)),
            pl.BlockSpec((1, _BC), lambda p, b: (0, b)),
            pl.BlockSpec((32, 4), lambda p, b: (0, 0)),
            pl.BlockSpec((32, 1), lambda p, b: (0, 0)),
            pl.BlockSpec((16, 32), lambda p, b: (0, 0)),
            pl.BlockSpec((16, 1), lambda p, b: (0, 0)),
            pl.BlockSpec((3, 16), lambda p, b: (0, 0)),
            pl.BlockSpec((3, 1), lambda p, b: (0, 0)),
        ],
        out_specs=[
            pl.BlockSpec((3, _BC), lambda p, b: (0, b)),
            pl.BlockSpec((1, _BC), lambda p, b: (0, b)),
            pl.BlockSpec(memory_space=pltpu.SMEM),
        ],
        out_shape=[
            jax.ShapeDtypeStruct((3, NP), jnp.float32),
            jax.ShapeDtypeStruct((1, NP), jnp.int32),
            jax.ShapeDtypeStruct((1, 3), jnp.int32),
        ],
        scratch_shapes=[
            pltpu.VMEM((4, NP), jnp.float32),
            pltpu.SMEM((2,), jnp.float32),
        ],
    )(addsT, mx, tblT, rl2d, w1t, b1c, w2t, b2c, w3t, b3c)


def kernel(x, uncertainty, edge_index, edge_lengths, refinement_level,
           W1, b1, W2, b2, W3, b3):
    f32, i32 = jnp.float32, jnp.int32
    src = edge_index[0]
    dst = edge_index[1]
    pad_n = NEP - N_EDGES
    pad_idx = (N_NODES +
               (jnp.arange(pad_n, dtype=i32) % (NP - N_NODES))).astype(i32)
    src_p = jnp.concatenate([src, pad_idx]).reshape(EROWS, 128)
    dst_p = jnp.concatenate([dst, pad_idx]).reshape(EROWS, 128)
    len_p = jnp.concatenate(
        [edge_lengths, jnp.ones((pad_n,), f32)]).reshape(EROWS, 128)
    xt = jnp.pad(x.T, ((0, 0), (0, NP - N_NODES)))
    ut = jnp.pad(uncertainty.T, ((0, 0), (0, NP - N_NODES)))

    tblT = _pre(xt, ut)
    adds, maxp, _ = _sc_call(tblT[0], tblT[1], tblT[2], tblT[3],
                             src_p, dst_p, len_p)
    adds = adds.reshape(NC, 3, NP)
    mx = maxp.reshape(NW, NP)
    rl2d = jnp.pad(refinement_level, (0, NP - N_NODES),
                   constant_values=1).reshape(1, NP)
    probsT, dec, cnt = _post(adds, mx, tblT, rl2d,
                             W1.T, b1.reshape(32, 1),
                             W2.T, b2.reshape(16, 1),
                             W3.T, b3.reshape(3, 1))
    return (x, edge_index, dec[0, :N_NODES], probsT[:, :N_NODES].T,
            tblT[3, :N_NODES], cnt[0, 0], cnt[0, 1], cnt[0, 2])


# speedup vs baseline: 60.1212x; 1.1162x over previous
import functools

import jax
import jax.numpy as jnp
from jax import lax
from jax.experimental import pallas as pl
from jax.experimental.pallas import tpu as pltpu
from jax.experimental.pallas import tpu_sc as plsc

N_NODES = 100000
NP = 100352
N_EDGES = 3200000
NEP = 3211264
EROWS = NEP // 128
NC, NS = 2, 16
NW = NC * NS
RPT = EROWS // NW
K = 8
CHUNKS = RPT // K
SLICE = NP // NS
NP2 = NP // 2
MAXL = 3


def _pre_body(xt_ref, ut_ref, tbl_ref):
    u = ut_ref[...]
    tbl_ref[0:3, :] = xt_ref[...]
    tbl_ref[3:4, :] = jnp.sqrt(jnp.sum(u * u, axis=0, keepdims=True))


def _pre(xt, ut):
    bc = 2048
    return pl.pallas_call(
        _pre_body,
        grid=(NP // bc,),
        in_specs=[pl.BlockSpec((3, bc), lambda b: (0, b)),
                  pl.BlockSpec((4, bc), lambda b: (0, b))],
        out_specs=pl.BlockSpec((4, bc), lambda b: (0, b)),
        out_shape=jax.ShapeDtypeStruct((4, NP), jnp.float32),
    )(xt, ut)


def _sc_body(t0_hbm, t1_hbm, t2_hbm, t3_hbm, src_hbm, dst_hbm, len_hbm,
             adds_hbm, maxp_hbm, sn_hbm, sv_hbm,
             src_v, dst_v, len_v, xs0_v, xs1_v, xs2_v, uns_v,
             xd0_v, xd1_v, xd2_v, und_v, grad_v, el_v, ones_v,
             snb_v, svb_v, maxacc, nbuf, zb,
             tb0, tb1, tb2, tb3, acc_g, acc_e, acc_d, sem):
    cid = lax.axis_index("c")
    sid = lax.axis_index("s")
    wid = cid * NS + sid

    iota = lax.iota(jnp.int32, 16)
    zf = jnp.zeros((16,), jnp.float32)
    onef = jnp.ones((16,), jnp.float32)

    nbuf[pl.ds(16, 16)] = jnp.full((16,), -1, jnp.int32)

    def _ones(i, carry):
        ones_v[pl.ds(i * 16, 16)] = onef
        return carry
    lax.fori_loop(0, 64, _ones, 0)

    def _zzb(i, carry):
        zb[pl.ds(i * 16, 16)] = zf
        return carry
    lax.fori_loop(0, 64, _zzb, 0)

    for acc in (acc_g, acc_e, acc_d):
        def _zacc(c, carry, acc=acc):
            col0 = sid * SLICE + c * 896
            pltpu.sync_copy(zb.at[pl.ds(0, 896)], acc.at[pl.ds(col0, 896)])
            return carry
        lax.fori_loop(0, 7, _zacc, 0)

    for th, tb in zip((t0_hbm, t1_hbm, t2_hbm, t3_hbm), (tb0, tb1, tb2, tb3)):
        def _stage(c, carry, th=th, tb=tb):
            col0 = sid * SLICE + c * 896
            pltpu.sync_copy(th.at[pl.ds(col0, 896)], zb.at[pl.ds(0, 896)])
            pltpu.sync_copy(zb.at[pl.ds(0, 896)], tb.at[pl.ds(col0, 896)])
            return carry
        lax.fori_loop(0, 7, _stage, 0)
    plsc.subcore_barrier()

    base_e = wid * RPT * 128

    def _chunk(cn, carry):
        e0 = base_e + cn * 1024
        pltpu.sync_copy(src_hbm.at[pl.ds(e0, 1024)], src_v)
        pltpu.sync_copy(dst_hbm.at[pl.ds(e0, 1024)], dst_v)
        pltpu.sync_copy(len_hbm.at[pl.ds(e0, 1024)], len_v)
        descs = []
        for tb, dv in ((tb0, xs0_v), (tb1, xs1_v), (tb2, xs2_v),
                       (tb3, uns_v)):
            descs.append(pltpu.async_copy(tb.at[src_v], dv, sem))
        for tb, dv in ((tb0, xd0_v), (tb1, xd1_v), (tb2, xd2_v),
                       (tb3, und_v)):
            descs.append(pltpu.async_copy(tb.at[dst_v], dv, sem))
        for d in descs:
            d.wait()

        def _grp(g, inner):
            k0 = g * 16
            sl = pl.ds(k0, 16)
            uns = uns_v[sl]
            und = und_v[sl]
            dx = xd0_v[sl] - xs0_v[sl]
            dy = xd1_v[sl] - xs1_v[sl]
            dz = xd2_v[sl] - xs2_v[sl]
            d2 = dx * dx + dy * dy + dz * dz
            z = lax.bitcast_convert_type(
                jnp.int32(0x5F3759DF) -
                (lax.bitcast_convert_type(d2, jnp.int32) >> 1), jnp.float32)
            z = z * (1.5 - 0.5 * d2 * z * z)
            z = z * (1.5 - 0.5 * d2 * z * z)
            z = z * (1.5 - 0.5 * d2 * z * z)
            el_v[sl] = d2 * z
            grad_v[sl] = jnp.abs(und - uns) / (len_v[sl] + 1e-8)
            s16 = src_v[sl]
            ub = lax.bitcast_convert_type(und, jnp.uint32)
            key = (s16.astype(jnp.uint32) << 15) | (ub >> 16)
            sk, sv = plsc.sort_key_val(key, und)
            sn = lax.shift_right_logical(
                sk, jnp.uint32(15)).astype(jnp.int32)
            nbuf[pl.ds(0, 16)] = sn
            nxt = plsc.load_gather(nbuf, [iota + 1])
            last = (sn != nxt) | (iota == 15)
            snb_v[sl] = jnp.where(last, sn, -1)
            svb_v[sl] = sv
            return inner
        lax.fori_loop(0, 64, _grp, 0)

        pltpu.sync_copy(snb_v, sn_hbm.at[pl.ds(e0, 1024)])
        pltpu.sync_copy(svb_v, sv_hbm.at[pl.ds(e0, 1024)])
        pltpu.sync_copy(grad_v, acc_g.at[src_v], add=True)
        pltpu.sync_copy(el_v, acc_e.at[src_v], add=True)
        pltpu.sync_copy(ones_v, acc_d.at[src_v], add=True)
        return carry
    lax.fori_loop(0, CHUNKS, _chunk, 0)

    for h in range(2):
        def _zm(i, carry):
            maxacc[pl.ds(i * 16, 16)] = zf
            return carry
        lax.fori_loop(0, NP2 // 16, _zm, 0)
        lo = h * NP2

        def _mchunk(cn, carry, lo=lo):
            e0 = base_e + cn * 1024
            pltpu.sync_copy(sn_hbm.at[pl.ds(e0, 1024)], src_v)
            pltpu.sync_copy(sv_hbm.at[pl.ds(e0, 1024)], len_v)

            def _mgrp(g, inner):
                sl = pl.ds(g * 16, 16)
                sn = src_v[sl]
                sv = len_v[sl]
                m = (sn >= lo) & (sn < lo + NP2)
                si = jnp.where(m, sn - lo, 0)
                cur = plsc.load_gather(maxacc, [si], mask=m)
                plsc.store_scatter(maxacc, [si], jnp.maximum(cur, sv),
                                   mask=m)
                return inner
            lax.fori_loop(0, 64, _mgrp, 0)
            return carry
        lax.fori_loop(0, CHUNKS, _mchunk, 0)
        pltpu.sync_copy(maxacc, maxp_hbm.at[pl.ds(wid * NP + lo, NP2)])

    plsc.subcore_barrier()

    for a, acc in enumerate((acc_g, acc_e, acc_d)):
        def _out(c, carry, a=a, acc=acc):
            col0 = sid * SLICE + c * 896
            pltpu.sync_copy(acc.at[pl.ds(col0, 896)], zb.at[pl.ds(0, 896)])
            pltpu.sync_copy(zb.at[pl.ds(0, 896)],
                            adds_hbm.at[pl.ds((cid * 3 + a) * NP + col0,
                                              896)])
            return carry
        lax.fori_loop(0, 7, _out, 0)


_sc_mesh = plsc.VectorSubcoreMesh(core_axis_name="c", subcore_axis_name="s",
                                  num_cores=NC, num_subcores=NS)
_sc_call = functools.partial(
    pl.kernel,
    out_type=[jax.ShapeDtypeStruct((NC * 3 * NP,), jnp.float32),
              jax.ShapeDtypeStruct((NW * NP,), jnp.float32),
              jax.ShapeDtypeStruct((NEP,), jnp.int32),
              jax.ShapeDtypeStruct((NEP,), jnp.float32)],
    mesh=_sc_mesh,
    compiler_params=pltpu.CompilerParams(needs_layout_passes=False),
    scratch_types=[
        pltpu.VMEM((1024,), jnp.int32),
        pltpu.VMEM((1024,), jnp.int32),
        pltpu.VMEM((1024,), jnp.float32),
        pltpu.VMEM((1024,), jnp.float32),
        pltpu.VMEM((1024,), jnp.float32),
        pltpu.VMEM((1024,), jnp.float32),
        pltpu.VMEM((1024,), jnp.float32),
        pltpu.VMEM((1024,), jnp.float32),
        pltpu.VMEM((1024,), jnp.float32),
        pltpu.VMEM((1024,), jnp.float32),
        pltpu.VMEM((1024,), jnp.float32),
        pltpu.VMEM((1024,), jnp.float32),
        pltpu.VMEM((1024,), jnp.float32),
        pltpu.VMEM((1024,), jnp.float32),
        pltpu.VMEM((1024,), jnp.int32),
        pltpu.VMEM((1024,), jnp.float32),
        pltpu.VMEM((NP2,), jnp.float32),
        pltpu.VMEM((32,), jnp.int32),
        pltpu.VMEM((1024,), jnp.float32),
        pltpu.VMEM_SHARED((NP,), jnp.float32),
        pltpu.VMEM_SHARED((NP,), jnp.float32),
        pltpu.VMEM_SHARED((NP,), jnp.float32),
        pltpu.VMEM_SHARED((NP,), jnp.float32),
        pltpu.VMEM_SHARED((NP,), jnp.float32),
        pltpu.VMEM_SHARED((NP,), jnp.float32),
        pltpu.VMEM_SHARED((NP,), jnp.float32),
        pltpu.SemaphoreType.DMA,
    ],
)(_sc_body)


_BC = 512


def _post_body(a_ref, mx_ref, tbl_ref, rl_ref, w1_ref, b1_ref, w2_ref,
               b2_ref, w3_ref, b3_ref, probs_ref, dec_ref, cnt_ref,
               feat, cmm):
    p = pl.program_id(0)
    b = pl.program_id(1)
    col0 = b * _BC
    cols = col0 + lax.broadcasted_iota(jnp.int32, (1, _BC), 1)
    real = cols < N_NODES

    @pl.when(p == 0)
    def _phase0():
        a0 = a_ref[0]
        a1 = a_ref[1]
        deg = a0[2:3] + a1[2:3]
        grad = (a0[0:1] + a1[0:1]) / (deg + 1e-8)
        curv = (a0[1:2] + a1[1:2]) / (deg + 1e-8)
        nmax = jnp.max(mx_ref[...], axis=0, keepdims=True)
        feat[0:1, pl.ds(col0, _BC)] = tbl_ref[3:4, :]
        feat[1:2, pl.ds(col0, _BC)] = grad
        feat[2:3, pl.ds(col0, _BC)] = curv
        feat[3:4, pl.ds(col0, _BC)] = nmax

        @pl.when(b == 0)
        def _init():
            cmm[0] = jnp.float32(3.0e38)
            cmm[1] = jnp.float32(-3.0e38)
        bmin = jnp.min(jnp.where(real, curv, jnp.float32(3.0e38)))
        bmax = jnp.max(jnp.where(real, curv, jnp.float32(-3.0e38)))
        cmm[0] = jnp.minimum(cmm[0], bmin)
        cmm[1] = jnp.maximum(cmm[1], bmax)

    @pl.when(p == 1)
    def _phase1():
        f = feat[:, pl.ds(col0, _BC)]
        curv = (f[2:3] - cmm[0]) / (cmm[1] - cmm[0] + 1e-8)
        feats = jnp.concatenate([f[0:2], curv, f[3:4]], axis=0)
        h = jnp.dot(w1_ref[...], feats,
                    preferred_element_type=jnp.float32) + b1_ref[...]
        h = jnp.maximum(h, 0.0)
        h = jnp.dot(w2_ref[...], h,
                    preferred_element_type=jnp.float32) + b2_ref[...]
        h = jnp.maximum(h, 0.0)
        lg = jnp.dot(w3_ref[...], h,
                     preferred_element_type=jnp.float32) + b3_ref[...]
        m = jnp.max(lg, axis=0, keepdims=True)
        e = jnp.exp(lg - m)
        probs_ref[...] = e / jnp.sum(e, axis=0, keepdims=True)
        l0 = lg[0:1]
        l1 = lg[1:2]
        l2 = lg[2:3]
        d = jnp.where(l1 > l0, 1, 0)
        d = jnp.where(l2 > jnp.maximum(l0, l1), 2, d)
        rl = rl_ref[...]
        d = jnp.where((rl >= MAXL) & (d == 0), 1, d)
        d = jnp.where((rl <= 0) & (d == 2), 1, d)
        d = d.astype(jnp.int32)
        dec_ref[...] = d

        @pl.when(b == 0)
        def _init():
            cnt_ref[0, 0] = 0
            cnt_ref[0, 1] = 0
            cnt_ref[0, 2] = 0
        cnt_ref[0, 0] += jnp.sum(jnp.where((d == 0) & real, 1, 0))
        cnt_ref[0, 1] += jnp.sum(jnp.where((d == 1) & real, 1, 0))
        cnt_ref[0, 2] += jnp.sum(jnp.where((d == 2) & real, 1, 0))


def _post(addsT, mx, tblT, rl2d, w1t, b1c, w2t, b2c, w3t, b3c):
    nb = NP // _BC
    return pl.pallas_call(
        _post_body,
        grid=(2, nb),
        in_specs=[
            pl.BlockSpec((2, 3, _BC), lambda p, b: (0, 0, b)),
            pl.BlockSpec((NW, _BC), lambda p, b: (0, b)),
            pl.BlockSpec((4, _BC), lambda p, b: (0, b)),
            pl.BlockSpec((1, _BC), lambda p, b: (0, b)),
            pl.BlockSpec((32, 4), lambda p, b: (0, 0)),
            pl.BlockSpec((32, 1), lambda p, b: (0, 0)),
            pl.BlockSpec((16, 32), lambda p, b: (0, 0)),
            pl.BlockSpec((16, 1), lambda p, b: (0, 0)),
            pl.BlockSpec((3, 16), lambda p, b: (0, 0)),
            pl.BlockSpec((3, 1), lambda p, b: (0, 0)),
        ],
        out_specs=[
            pl.BlockSpec((3, _BC), lambda p, b: (0, b)),
            pl.BlockSpec((1, _BC), lambda p, b: (0, b)),
            pl.BlockSpec(memory_space=pltpu.SMEM),
        ],
        out_shape=[
            jax.ShapeDtypeStruct((3, NP), jnp.float32),
            jax.ShapeDtypeStruct((1, NP), jnp.int32),
            jax.ShapeDtypeStruct((1, 3), jnp.int32),
        ],
        scratch_shapes=[
            pltpu.VMEM((4, NP), jnp.float32),
            pltpu.SMEM((2,), jnp.float32),
        ],
    )(addsT, mx, tblT, rl2d, w1t, b1c, w2t, b2c, w3t, b3c)


def kernel(x, uncertainty, edge_index, edge_lengths, refinement_level,
           W1, b1, W2, b2, W3, b3):
    f32, i32 = jnp.float32, jnp.int32
    src = edge_index[0]
    dst = edge_index[1]
    pad_n = NEP - N_EDGES
    pad_idx = (N_NODES +
               (jnp.arange(pad_n, dtype=i32) % (NP - N_NODES))).astype(i32)
    src_p = jnp.concatenate([src, pad_idx])
    dst_p = jnp.concatenate([dst, pad_idx])
    len_p = jnp.concatenate([edge_lengths, jnp.ones((pad_n,), f32)])
    xt = jnp.pad(x.T, ((0, 0), (0, NP - N_NODES)))
    ut = jnp.pad(uncertainty.T, ((0, 0), (0, NP - N_NODES)))

    tblT = _pre(xt, ut)
    adds, maxp, _sn, _sv = _sc_call(tblT[0], tblT[1], tblT[2], tblT[3],
                                    src_p, dst_p, len_p)
    adds = adds.reshape(NC, 3, NP)
    mx = maxp.reshape(NW, NP)
    rl2d = jnp.pad(refinement_level, (0, NP - N_NODES),
                   constant_values=1).reshape(1, NP)
    probsT, dec, cnt = _post(adds, mx, tblT, rl2d,
                             W1.T, b1.reshape(32, 1),
                             W2.T, b2.reshape(16, 1),
                             W3.T, b3.reshape(3, 1))
    return (x, edge_index, dec[0, :N_NODES], probsT[:, :N_NODES].T,
            tblT[3, :N_NODES], cnt[0, 0], cnt[0, 1], cnt[0, 2])


# speedup vs baseline: 66.1270x; 1.0999x over previous
import functools

import jax
import jax.numpy as jnp
from jax import lax
from jax.experimental import pallas as pl
from jax.experimental.pallas import tpu as pltpu
from jax.experimental.pallas import tpu_sc as plsc

N_NODES = 100000
NP = 100352
N_EDGES = 3200000
NEP = 3211264
EROWS = NEP // 128
NC, NS = 2, 16
NW = NC * NS
RPT = EROWS // NW
K = 8
CHUNKS = RPT // K
SLICE = NP // NS
NP2 = NP // 2
MAXL = 3


def _pre_body(xt_ref, ut_ref, tbl_ref):
    u = ut_ref[...]
    tbl_ref[0:3, :] = xt_ref[...]
    tbl_ref[3:4, :] = jnp.sqrt(jnp.sum(u * u, axis=0, keepdims=True))


def _pre(xt, ut):
    bc = 2048
    return pl.pallas_call(
        _pre_body,
        grid=(NP // bc,),
        in_specs=[pl.BlockSpec((3, bc), lambda b: (0, b)),
                  pl.BlockSpec((4, bc), lambda b: (0, b))],
        out_specs=pl.BlockSpec((4, bc), lambda b: (0, b)),
        out_shape=jax.ShapeDtypeStruct((4, NP), jnp.float32),
    )(xt, ut)


def _sc_body(t0_hbm, t1_hbm, t2_hbm, t3_hbm, src_hbm, dst_hbm, len_hbm,
             adds_hbm, maxp_hbm, sn_hbm, sv_hbm,
             src_v, dst_v, len_v, xs0_v, xs1_v, xs2_v, uns_v,
             xd0_v, xd1_v, xd2_v, und_v, grad_v, el_v, ones_v,
             snb_v, svb_v, maxacc, nbuf, zb,
             tb0, tb1, tb2, tb3, acc_g, acc_e, acc_d, sem):
    cid = lax.axis_index("c")
    sid = lax.axis_index("s")
    wid = cid * NS + sid

    iota = lax.iota(jnp.int32, 16)
    zf = jnp.zeros((16,), jnp.float32)
    onef = jnp.ones((16,), jnp.float32)

    def _nsent(i, carry):
        nbuf[pl.ds(i * 16, 16)] = jnp.full((16,), -1, jnp.int32)
        return carry
    lax.fori_loop(0, 128, _nsent, 0)

    def _ones(i, carry):
        ones_v[pl.ds(i * 16, 16)] = onef
        return carry
    lax.fori_loop(0, 64, _ones, 0)

    def _zzb(i, carry):
        zb[pl.ds(i * 16, 16)] = zf
        return carry
    lax.fori_loop(0, 64, _zzb, 0)

    for acc in (acc_g, acc_e, acc_d):
        def _zacc(c, carry, acc=acc):
            col0 = sid * SLICE + c * 896
            pltpu.sync_copy(zb.at[pl.ds(0, 896)], acc.at[pl.ds(col0, 896)])
            return carry
        lax.fori_loop(0, 7, _zacc, 0)

    for th, tb in zip((t0_hbm, t1_hbm, t2_hbm, t3_hbm), (tb0, tb1, tb2, tb3)):
        def _stage(c, carry, th=th, tb=tb):
            col0 = sid * SLICE + c * 896
            pltpu.sync_copy(th.at[pl.ds(col0, 896)], zb.at[pl.ds(0, 896)])
            pltpu.sync_copy(zb.at[pl.ds(0, 896)], tb.at[pl.ds(col0, 896)])
            return carry
        lax.fori_loop(0, 7, _stage, 0)
    plsc.subcore_barrier()

    base_e = wid * RPT * 128

    def _chunk(cn, carry):
        e0 = base_e + cn * 1024
        pltpu.sync_copy(src_hbm.at[pl.ds(e0, 1024)], src_v)
        pltpu.sync_copy(dst_hbm.at[pl.ds(e0, 1024)], dst_v)
        pltpu.sync_copy(len_hbm.at[pl.ds(e0, 1024)], len_v)
        descs = []
        for tb, dv in ((tb0, xs0_v), (tb1, xs1_v), (tb2, xs2_v),
                       (tb3, uns_v)):
            descs.append(pltpu.async_copy(tb.at[src_v], dv, sem))
        for tb, dv in ((tb0, xd0_v), (tb1, xd1_v), (tb2, xd2_v),
                       (tb3, und_v)):
            descs.append(pltpu.async_copy(tb.at[dst_v], dv, sem))
        for d in descs:
            d.wait()

        @plsc.parallel_loop(0, 64, unroll=4)
        def _grp(g):
            k0 = g * 16
            sl = pl.ds(k0, 16)
            uns = uns_v[sl]
            und = und_v[sl]
            dx = xd0_v[sl] - xs0_v[sl]
            dy = xd1_v[sl] - xs1_v[sl]
            dz = xd2_v[sl] - xs2_v[sl]
            d2 = dx * dx + dy * dy + dz * dz
            z = lax.bitcast_convert_type(
                jnp.int32(0x5F3759DF) -
                (lax.bitcast_convert_type(d2, jnp.int32) >> 1), jnp.float32)
            z = z * (1.5 - 0.5 * d2 * z * z)
            z = z * (1.5 - 0.5 * d2 * z * z)
            z = z * (1.5 - 0.5 * d2 * z * z)
            el_v[sl] = d2 * z
            grad_v[sl] = jnp.abs(und - uns) / (len_v[sl] + 1e-8)
            s16 = src_v[sl]
            ub = lax.bitcast_convert_type(und, jnp.uint32)
            key = (s16.astype(jnp.uint32) << 15) | (ub >> 16)
            sk, sv = plsc.sort_key_val(key, und)
            sn = lax.shift_right_logical(
                sk, jnp.uint32(15)).astype(jnp.int32)
            slot = g * 32
            nbuf[pl.ds(slot, 16)] = sn
            nxt = plsc.load_gather(nbuf, [slot + iota + 1])
            last = (sn != nxt) | (iota == 15)
            snb_v[sl] = jnp.where(last, sn, -1)
            svb_v[sl] = sv

        pltpu.sync_copy(snb_v, sn_hbm.at[pl.ds(e0, 1024)])
        pltpu.sync_copy(svb_v, sv_hbm.at[pl.ds(e0, 1024)])
        pltpu.sync_copy(grad_v, acc_g.at[src_v], add=True)
        pltpu.sync_copy(el_v, acc_e.at[src_v], add=True)
        pltpu.sync_copy(ones_v, acc_d.at[src_v], add=True)
        return carry
    lax.fori_loop(0, CHUNKS, _chunk, 0)

    for h in range(2):
        def _zm(i, carry):
            maxacc[pl.ds(i * 16, 16)] = zf
            return carry
        lax.fori_loop(0, NP2 // 16, _zm, 0)
        lo = h * NP2

        def _mchunk(cn, carry, lo=lo):
            e0 = base_e + cn * 1024
            pltpu.sync_copy(sn_hbm.at[pl.ds(e0, 1024)], src_v)
            pltpu.sync_copy(sv_hbm.at[pl.ds(e0, 1024)], len_v)

            def _mgrp(g, inner):
                sl = pl.ds(g * 16, 16)
                sn = src_v[sl]
                sv = len_v[sl]
                m = (sn >= lo) & (sn < lo + NP2)
                si = jnp.where(m, sn - lo, 0)
                cur = plsc.load_gather(maxacc, [si], mask=m)
                plsc.store_scatter(maxacc, [si], jnp.maximum(cur, sv),
                                   mask=m)
                return inner
            lax.fori_loop(0, 64, _mgrp, 0, unroll=4)
            return carry
        lax.fori_loop(0, CHUNKS, _mchunk, 0)
        pltpu.sync_copy(maxacc, maxp_hbm.at[pl.ds(wid * NP + lo, NP2)])

    plsc.subcore_barrier()

    for a, acc in enumerate((acc_g, acc_e, acc_d)):
        def _out(c, carry, a=a, acc=acc):
            col0 = sid * SLICE + c * 896
            pltpu.sync_copy(acc.at[pl.ds(col0, 896)], zb.at[pl.ds(0, 896)])
            pltpu.sync_copy(zb.at[pl.ds(0, 896)],
                            adds_hbm.at[pl.ds((cid * 3 + a) * NP + col0,
                                              896)])
            return carry
        lax.fori_loop(0, 7, _out, 0)


_sc_mesh = plsc.VectorSubcoreMesh(core_axis_name="c", subcore_axis_name="s",
                                  num_cores=NC, num_subcores=NS)
_sc_call = functools.partial(
    pl.kernel,
    out_type=[jax.ShapeDtypeStruct((NC * 3 * NP,), jnp.float32),
              jax.ShapeDtypeStruct((NW * NP,), jnp.float32),
              jax.ShapeDtypeStruct((NEP,), jnp.int32),
              jax.ShapeDtypeStruct((NEP,), jnp.float32)],
    mesh=_sc_mesh,
    compiler_params=pltpu.CompilerParams(needs_layout_passes=False),
    scratch_types=[
        pltpu.VMEM((1024,), jnp.int32),
        pltpu.VMEM((1024,), jnp.int32),
        pltpu.VMEM((1024,), jnp.float32),
        pltpu.VMEM((1024,), jnp.float32),
        pltpu.VMEM((1024,), jnp.float32),
        pltpu.VMEM((1024,), jnp.float32),
        pltpu.VMEM((1024,), jnp.float32),
        pltpu.VMEM((1024,), jnp.float32),
        pltpu.VMEM((1024,), jnp.float32),
        pltpu.VMEM((1024,), jnp.float32),
        pltpu.VMEM((1024,), jnp.float32),
        pltpu.VMEM((1024,), jnp.float32),
        pltpu.VMEM((1024,), jnp.float32),
        pltpu.VMEM((1024,), jnp.float32),
        pltpu.VMEM((1024,), jnp.int32),
        pltpu.VMEM((1024,), jnp.float32),
        pltpu.VMEM((NP2,), jnp.float32),
        pltpu.VMEM((2048,), jnp.int32),
        pltpu.VMEM((1024,), jnp.float32),
        pltpu.VMEM_SHARED((NP,), jnp.float32),
        pltpu.VMEM_SHARED((NP,), jnp.float32),
        pltpu.VMEM_SHARED((NP,), jnp.float32),
        pltpu.VMEM_SHARED((NP,), jnp.float32),
        pltpu.VMEM_SHARED((NP,), jnp.float32),
        pltpu.VMEM_SHARED((NP,), jnp.float32),
        pltpu.VMEM_SHARED((NP,), jnp.float32),
        pltpu.SemaphoreType.DMA,
    ],
)(_sc_body)


_BC = 512


def _post_body(a_ref, mx_ref, tbl_ref, rl_ref, w1_ref, b1_ref, w2_ref,
               b2_ref, w3_ref, b3_ref, probs_ref, dec_ref, cnt_ref,
               feat, cmm):
    p = pl.program_id(0)
    b = pl.program_id(1)
    col0 = b * _BC
    cols = col0 + lax.broadcasted_iota(jnp.int32, (1, _BC), 1)
    real = cols < N_NODES

    @pl.when(p == 0)
    def _phase0():
        a0 = a_ref[0]
        a1 = a_ref[1]
        deg = a0[2:3] + a1[2:3]
        grad = (a0[0:1] + a1[0:1]) / (deg + 1e-8)
        curv = (a0[1:2] + a1[1:2]) / (deg + 1e-8)
        nmax = jnp.max(mx_ref[...], axis=0, keepdims=True)
        feat[0:1, pl.ds(col0, _BC)] = tbl_ref[3:4, :]
        feat[1:2, pl.ds(col0, _BC)] = grad
        feat[2:3, pl.ds(col0, _BC)] = curv
        feat[3:4, pl.ds(col0, _BC)] = nmax

        @pl.when(b == 0)
        def _init():
            cmm[0] = jnp.float32(3.0e38)
            cmm[1] = jnp.float32(-3.0e38)
        bmin = jnp.min(jnp.where(real, curv, jnp.float32(3.0e38)))
        bmax = jnp.max(jnp.where(real, curv, jnp.float32(-3.0e38)))
        cmm[0] = jnp.minimum(cmm[0], bmin)
        cmm[1] = jnp.maximum(cmm[1], bmax)

    @pl.when(p == 1)
    def _phase1():
        f = feat[:, pl.ds(col0, _BC)]
        curv = (f[2:3] - cmm[0]) / (cmm[1] - cmm[0] + 1e-8)
        feats = jnp.concatenate([f[0:2], curv, f[3:4]], axis=0)
        h = jnp.dot(w1_ref[...], feats,
                    preferred_element_type=jnp.float32) + b1_ref[...]
        h = jnp.maximum(h, 0.0)
        h = jnp.dot(w2_ref[...], h,
                    preferred_element_type=jnp.float32) + b2_ref[...]
        h = jnp.maximum(h, 0.0)
        lg = jnp.dot(w3_ref[...], h,
                     preferred_element_type=jnp.float32) + b3_ref[...]
        m = jnp.max(lg, axis=0, keepdims=True)
        e = jnp.exp(lg - m)
        probs_ref[...] = e / jnp.sum(e, axis=0, keepdims=True)
        l0 = lg[0:1]
        l1 = lg[1:2]
        l2 = lg[2:3]
        d = jnp.where(l1 > l0, 1, 0)
        d = jnp.where(l2 > jnp.maximum(l0, l1), 2, d)
        rl = rl_ref[...]
        d = jnp.where((rl >= MAXL) & (d == 0), 1, d)
        d = jnp.where((rl <= 0) & (d == 2), 1, d)
        d = d.astype(jnp.int32)
        dec_ref[...] = d

        @pl.when(b == 0)
        def _init():
            cnt_ref[0, 0] = 0
            cnt_ref[0, 1] = 0
            cnt_ref[0, 2] = 0
        cnt_ref[0, 0] += jnp.sum(jnp.where((d == 0) & real, 1, 0))
        cnt_ref[0, 1] += jnp.sum(jnp.where((d == 1) & real, 1, 0))
        cnt_ref[0, 2] += jnp.sum(jnp.where((d == 2) & real, 1, 0))


def _post(addsT, mx, tblT, rl2d, w1t, b1c, w2t, b2c, w3t, b3c):
    nb = NP // _BC
    return pl.pallas_call(
        _post_body,
        grid=(2, nb),
        in_specs=[
            pl.BlockSpec((2, 3, _BC), lambda p, b: (0, 0, b)),
            pl.BlockSpec((NW, _BC), lambda p, b: (0, b)),
            pl.BlockSpec((4, _BC), lambda p, b: (0, b)),
            pl.BlockSpec((1, _BC), lambda p, b: (0, b)),
            pl.BlockSpec((32, 4), lambda p, b: (0, 0)),
            pl.BlockSpec((32, 1), lambda p, b: (0, 0)),
            pl.BlockSpec((16, 32), lambda p, b: (0, 0)),
            pl.BlockSpec((16, 1), lambda p, b: (0, 0)),
            pl.BlockSpec((3, 16), lambda p, b: (0, 0)),
            pl.BlockSpec((3, 1), lambda p, b: (0, 0)),
        ],
        out_specs=[
            pl.BlockSpec((3, _BC), lambda p, b: (0, b)),
            pl.BlockSpec((1, _BC), lambda p, b: (0, b)),
            pl.BlockSpec(memory_space=pltpu.SMEM),
        ],
        out_shape=[
            jax.ShapeDtypeStruct((3, NP), jnp.float32),
            jax.ShapeDtypeStruct((1, NP), jnp.int32),
            jax.ShapeDtypeStruct((1, 3), jnp.int32),
        ],
        scratch_shapes=[
            pltpu.VMEM((4, NP), jnp.float32),
            pltpu.SMEM((2,), jnp.float32),
        ],
    )(addsT, mx, tblT, rl2d, w1t, b1c, w2t, b2c, w3t, b3c)


def kernel(x, uncertainty, edge_index, edge_lengths, refinement_level,
           W1, b1, W2, b2, W3, b3):
    f32, i32 = jnp.float32, jnp.int32
    src = edge_index[0]
    dst = edge_index[1]
    pad_n = NEP - N_EDGES
    pad_idx = (N_NODES +
               (jnp.arange(pad_n, dtype=i32) % (NP - N_NODES))).astype(i32)
    src_p = jnp.concatenate([src, pad_idx])
    dst_p = jnp.concatenate([dst, pad_idx])
    len_p = jnp.concatenate([edge_lengths, jnp.ones((pad_n,), f32)])
    xt = jnp.pad(x.T, ((0, 0), (0, NP - N_NODES)))
    ut = jnp.pad(uncertainty.T, ((0, 0), (0, NP - N_NODES)))

    tblT = _pre(xt, ut)
    adds, maxp, _sn, _sv = _sc_call(tblT[0], tblT[1], tblT[2], tblT[3],
                                    src_p, dst_p, len_p)
    adds = adds.reshape(NC, 3, NP)
    mx = maxp.reshape(NW, NP)
    rl2d = jnp.pad(refinement_level, (0, NP - N_NODES),
                   constant_values=1).reshape(1, NP)
    probsT, dec, cnt = _post(adds, mx, tblT, rl2d,
                             W1.T, b1.reshape(32, 1),
                             W2.T, b2.reshape(16, 1),
                             W3.T, b3.reshape(3, 1))
    return (x, edge_index, dec[0, :N_NODES], probsT[:, :N_NODES].T,
            tblT[3, :N_NODES], cnt[0, 0], cnt[0, 1], cnt[0, 2])


# speedup vs baseline: 93.9047x; 1.4201x over previous
import functools

import jax
import jax.numpy as jnp
from jax import lax
from jax.experimental import pallas as pl
from jax.experimental.pallas import tpu as pltpu
from jax.experimental.pallas import tpu_sc as plsc

N_NODES = 100000
NP = 100352
N_EDGES = 3200000
NEP = 3211264
EROWS = NEP // 128
NC, NS = 2, 16
NW = NC * NS
RPT = EROWS // NW
K = 8
CHUNKS = RPT // K
SLICE = NP // NS
NP2 = NP // 2
MAXL = 3


def _pre_body(xt_ref, ut_ref, tbl_ref):
    u = ut_ref[...]
    tbl_ref[0:3, :] = xt_ref[...]
    tbl_ref[3:4, :] = jnp.sqrt(jnp.sum(u * u, axis=0, keepdims=True))


def _pre(xt, ut):
    bc = 2048
    return pl.pallas_call(
        _pre_body,
        grid=(NP // bc,),
        in_specs=[pl.BlockSpec((3, bc), lambda b: (0, b)),
                  pl.BlockSpec((4, bc), lambda b: (0, b))],
        out_specs=pl.BlockSpec((4, bc), lambda b: (0, b)),
        out_shape=jax.ShapeDtypeStruct((4, NP), jnp.float32),
    )(xt, ut)


def _sc_body(t0_hbm, t1_hbm, t2_hbm, t3_hbm, src_hbm, dst_hbm, len_hbm,
             adds_hbm, maxp_hbm, sn_hbm, sv_hbm, *refs):
    (srcA, dstA, lenA, xs0A, xs1A, xs2A, unsA, xd0A, xd1A, xd2A, undA,
     srcB, dstB, lenB, xs0B, xs1B, xs2B, unsB, xd0B, xd1B, xd2B, undB,
     grad_v, el_v, ones_v, snb_v, svb_v, maxacc, nbuf, zb,
     tb0, tb1, tb2, tb3, acc_g, acc_e, acc_d, semA, semB) = refs
    bufA = (srcA, dstA, lenA, xs0A, xs1A, xs2A, unsA, xd0A, xd1A, xd2A,
            undA)
    bufB = (srcB, dstB, lenB, xs0B, xs1B, xs2B, unsB, xd0B, xd1B, xd2B,
            undB)
    cid = lax.axis_index("c")
    sid = lax.axis_index("s")
    wid = cid * NS + sid

    iota = lax.iota(jnp.int32, 16)
    zf = jnp.zeros((16,), jnp.float32)
    onef = jnp.ones((16,), jnp.float32)

    def _nsent(i, carry):
        nbuf[pl.ds(i * 16, 16)] = jnp.full((16,), -1, jnp.int32)
        return carry
    lax.fori_loop(0, 128, _nsent, 0)

    def _ones(i, carry):
        ones_v[pl.ds(i * 16, 16)] = onef
        return carry
    lax.fori_loop(0, 64, _ones, 0)

    def _zzb(i, carry):
        zb[pl.ds(i * 16, 16)] = zf
        return carry
    lax.fori_loop(0, 64, _zzb, 0)

    for acc in (acc_g, acc_e, acc_d):
        def _zacc(c, carry, acc=acc):
            col0 = sid * SLICE + c * 896
            pltpu.sync_copy(zb.at[pl.ds(0, 896)], acc.at[pl.ds(col0, 896)])
            return carry
        lax.fori_loop(0, 7, _zacc, 0)

    for th, tb in zip((t0_hbm, t1_hbm, t2_hbm, t3_hbm), (tb0, tb1, tb2, tb3)):
        def _stage(c, carry, th=th, tb=tb):
            col0 = sid * SLICE + c * 896
            pltpu.sync_copy(th.at[pl.ds(col0, 896)], zb.at[pl.ds(0, 896)])
            pltpu.sync_copy(zb.at[pl.ds(0, 896)], tb.at[pl.ds(col0, 896)])
            return carry
        lax.fori_loop(0, 7, _stage, 0)
    plsc.subcore_barrier()

    base_e = wid * RPT * 128

    def _load_fire(bufs, sem_, cn):
        e0 = base_e + cn * 1024
        pltpu.sync_copy(src_hbm.at[pl.ds(e0, 1024)], bufs[0])
        pltpu.sync_copy(dst_hbm.at[pl.ds(e0, 1024)], bufs[1])
        pltpu.sync_copy(len_hbm.at[pl.ds(e0, 1024)], bufs[2])
        ds_ = []
        for tb, dv in zip((tb0, tb1, tb2, tb3), bufs[3:7]):
            ds_.append(pltpu.async_copy(tb.at[bufs[0]], dv, sem_))
        for tb, dv in zip((tb0, tb1, tb2, tb3), bufs[7:11]):
            ds_.append(pltpu.async_copy(tb.at[bufs[1]], dv, sem_))
        return ds_

    def _drain_fake(bufs, sem_):
        for dv in bufs[3:11]:
            pltpu.make_async_copy(len_hbm.at[pl.ds(0, 1024)], dv,
                                  sem_).wait()

    def _compute(bufs, cn):
        srcv, dstv, lenv = bufs[0], bufs[1], bufs[2]
        x0s, x1s, x2s, unsv, x0d, x1d, x2d, undv = bufs[3:11]

        @plsc.parallel_loop(0, 64, unroll=4)
        def _grp(g):
            k0 = g * 16
            sl = pl.ds(k0, 16)
            uns = unsv[sl]
            und = undv[sl]
            dx = x0d[sl] - x0s[sl]
            dy = x1d[sl] - x1s[sl]
            dz = x2d[sl] - x2s[sl]
            d2 = dx * dx + dy * dy + dz * dz
            z = lax.bitcast_convert_type(
                jnp.int32(0x5F3759DF) -
                (lax.bitcast_convert_type(d2, jnp.int32) >> 1), jnp.float32)
            z = z * (1.5 - 0.5 * d2 * z * z)
            z = z * (1.5 - 0.5 * d2 * z * z)
            z = z * (1.5 - 0.5 * d2 * z * z)
            el_v[sl] = d2 * z
            grad_v[sl] = jnp.abs(und - uns) / (lenv[sl] + 1e-8)
            s16 = srcv[sl]
            ub = lax.bitcast_convert_type(und, jnp.uint32)
            key = (s16.astype(jnp.uint32) << 15) | (ub >> 16)
            sk, sv = plsc.sort_key_val(key, und)
            sn = lax.shift_right_logical(
                sk, jnp.uint32(15)).astype(jnp.int32)
            slot = g * 32
            nbuf[pl.ds(slot, 16)] = sn
            nxt = plsc.load_gather(nbuf, [slot + iota + 1])
            last = (sn != nxt) | (iota == 15)
            snb_v[sl] = jnp.where(last, sn, -1)
            svb_v[sl] = sv

        e0 = base_e + cn * 1024
        pltpu.sync_copy(snb_v, sn_hbm.at[pl.ds(e0, 1024)])
        pltpu.sync_copy(svb_v, sv_hbm.at[pl.ds(e0, 1024)])
        pltpu.sync_copy(grad_v, acc_g.at[srcv], add=True)
        pltpu.sync_copy(el_v, acc_e.at[srcv], add=True)
        pltpu.sync_copy(ones_v, acc_d.at[srcv], add=True)

    _load_fire(bufA, semA, 0)

    def _pair(t, carry):
        cn0 = 2 * t
        dB = _load_fire(bufB, semB, cn0 + 1)
        _drain_fake(bufA, semA)
        _compute(bufA, cn0)
        _load_fire(bufA, semA, (cn0 + 2) % CHUNKS)
        for d in dB:
            d.wait()
        _compute(bufB, cn0 + 1)
        return carry
    lax.fori_loop(0, CHUNKS // 2, _pair, 0)
    _drain_fake(bufA, semA)

    def _mfire(snb, svb, sem_, cn):
        e0 = base_e + cn * 1024
        d1 = pltpu.async_copy(sn_hbm.at[pl.ds(e0, 1024)], snb, sem_)
        d2 = pltpu.async_copy(sv_hbm.at[pl.ds(e0, 1024)], svb, sem_)
        return d1, d2

    def _mdrain(snb, svb, sem_):
        pltpu.make_async_copy(sn_hbm.at[pl.ds(0, 1024)], snb, sem_).wait()
        pltpu.make_async_copy(sv_hbm.at[pl.ds(0, 1024)], svb, sem_).wait()

    for h in range(2):
        def _zm(i, carry):
            maxacc[pl.ds(i * 16, 16)] = zf
            return carry
        lax.fori_loop(0, NP2 // 16, _zm, 0)
        lo = h * NP2

        def _mcomp(snb, svb, lo=lo):
            def _mgrp(g, inner):
                sl = pl.ds(g * 16, 16)
                sn = snb[sl]
                sv = svb[sl]
                m = (sn >= lo) & (sn < lo + NP2)
                si = jnp.where(m, sn - lo, 0)
                cur = plsc.load_gather(maxacc, [si], mask=m)
                plsc.store_scatter(maxacc, [si], jnp.maximum(cur, sv),
                                   mask=m)
                return inner
            lax.fori_loop(0, 64, _mgrp, 0, unroll=4)

        _mfire(srcA, lenA, semA, 0)

        def _mpair(t, carry):
            cn0 = 2 * t
            dB = _mfire(srcB, lenB, semB, cn0 + 1)
            _mdrain(srcA, lenA, semA)
            _mcomp(srcA, lenA)
            _mfire(srcA, lenA, semA, (cn0 + 2) % CHUNKS)
            for d in dB:
                d.wait()
            _mcomp(srcB, lenB)
            return carry
        lax.fori_loop(0, CHUNKS // 2, _mpair, 0)
        _mdrain(srcA, lenA, semA)
        pltpu.sync_copy(maxacc, maxp_hbm.at[pl.ds(wid * NP + lo, NP2)])

    plsc.subcore_barrier()

    for a, acc in enumerate((acc_g, acc_e, acc_d)):
        def _out(c, carry, a=a, acc=acc):
            col0 = sid * SLICE + c * 896
            pltpu.sync_copy(acc.at[pl.ds(col0, 896)], zb.at[pl.ds(0, 896)])
            pltpu.sync_copy(zb.at[pl.ds(0, 896)],
                            adds_hbm.at[pl.ds((cid * 3 + a) * NP + col0,
                                              896)])
            return carry
        lax.fori_loop(0, 7, _out, 0)


_sc_mesh = plsc.VectorSubcoreMesh(core_axis_name="c", subcore_axis_name="s",
                                  num_cores=NC, num_subcores=NS)
_sc_call = functools.partial(
    pl.kernel,
    out_type=[jax.ShapeDtypeStruct((NC * 3 * NP,), jnp.float32),
              jax.ShapeDtypeStruct((NW * NP,), jnp.float32),
              jax.ShapeDtypeStruct((NEP,), jnp.int32),
              jax.ShapeDtypeStruct((NEP,), jnp.float32)],
    mesh=_sc_mesh,
    compiler_params=pltpu.CompilerParams(needs_layout_passes=False),
    scratch_types=(
        [pltpu.VMEM((1024,), jnp.int32)] * 2 +
        [pltpu.VMEM((1024,), jnp.float32)] * 9 +
        [pltpu.VMEM((1024,), jnp.int32)] * 2 +
        [pltpu.VMEM((1024,), jnp.float32)] * 9 +
        [pltpu.VMEM((1024,), jnp.float32)] * 3 +
        [pltpu.VMEM((1024,), jnp.int32)] +
        [pltpu.VMEM((1024,), jnp.float32)] +
        [pltpu.VMEM((NP2,), jnp.float32)] +
        [pltpu.VMEM((2048,), jnp.int32)] +
        [pltpu.VMEM((1024,), jnp.float32)] +
        [pltpu.VMEM_SHARED((NP,), jnp.float32)] * 7 +
        [pltpu.SemaphoreType.DMA] * 2
    ),
)(_sc_body)


_BC = 512


def _post_body(a_ref, mx_ref, tbl_ref, rl_ref, w1_ref, b1_ref, w2_ref,
               b2_ref, w3_ref, b3_ref, probs_ref, dec_ref, cnt_ref,
               feat, cmm):
    p = pl.program_id(0)
    b = pl.program_id(1)
    col0 = b * _BC
    cols = col0 + lax.broadcasted_iota(jnp.int32, (1, _BC), 1)
    real = cols < N_NODES

    @pl.when(p == 0)
    def _phase0():
        a0 = a_ref[0]
        a1 = a_ref[1]
        deg = a0[2:3] + a1[2:3]
        grad = (a0[0:1] + a1[0:1]) / (deg + 1e-8)
        curv = (a0[1:2] + a1[1:2]) / (deg + 1e-8)
        nmax = jnp.max(mx_ref[...], axis=0, keepdims=True)
        feat[0:1, pl.ds(col0, _BC)] = tbl_ref[3:4, :]
        feat[1:2, pl.ds(col0, _BC)] = grad
        feat[2:3, pl.ds(col0, _BC)] = curv
        feat[3:4, pl.ds(col0, _BC)] = nmax

        @pl.when(b == 0)
        def _init():
            cmm[0] = jnp.float32(3.0e38)
            cmm[1] = jnp.float32(-3.0e38)
        bmin = jnp.min(jnp.where(real, curv, jnp.float32(3.0e38)))
        bmax = jnp.max(jnp.where(real, curv, jnp.float32(-3.0e38)))
        cmm[0] = jnp.minimum(cmm[0], bmin)
        cmm[1] = jnp.maximum(cmm[1], bmax)

    @pl.when(p == 1)
    def _phase1():
        f = feat[:, pl.ds(col0, _BC)]
        curv = (f[2:3] - cmm[0]) / (cmm[1] - cmm[0] + 1e-8)
        feats = jnp.concatenate([f[0:2], curv, f[3:4]], axis=0)
        h = jnp.dot(w1_ref[...], feats,
                    preferred_element_type=jnp.float32) + b1_ref[...]
        h = jnp.maximum(h, 0.0)
        h = jnp.dot(w2_ref[...], h,
                    preferred_element_type=jnp.float32) + b2_ref[...]
        h = jnp.maximum(h, 0.0)
        lg = jnp.dot(w3_ref[...], h,
                     preferred_element_type=jnp.float32) + b3_ref[...]
        m = jnp.max(lg, axis=0, keepdims=True)
        e = jnp.exp(lg - m)
        probs_ref[...] = e / jnp.sum(e, axis=0, keepdims=True)
        l0 = lg[0:1]
        l1 = lg[1:2]
        l2 = lg[2:3]
        d = jnp.where(l1 > l0, 1, 0)
        d = jnp.where(l2 > jnp.maximum(l0, l1), 2, d)
        rl = rl_ref[...]
        d = jnp.where((rl >= MAXL) & (d == 0), 1, d)
        d = jnp.where((rl <= 0) & (d == 2), 1, d)
        d = d.astype(jnp.int32)
        dec_ref[...] = d

        @pl.when(b == 0)
        def _init():
            cnt_ref[0, 0] = 0
            cnt_ref[0, 1] = 0
            cnt_ref[0, 2] = 0
        cnt_ref[0, 0] += jnp.sum(jnp.where((d == 0) & real, 1, 0))
        cnt_ref[0, 1] += jnp.sum(jnp.where((d == 1) & real, 1, 0))
        cnt_ref[0, 2] += jnp.sum(jnp.where((d == 2) & real, 1, 0))


def _post(addsT, mx, tblT, rl2d, w1t, b1c, w2t, b2c, w3t, b3c):
    nb = NP // _BC
    return pl.pallas_call(
        _post_body,
        grid=(2, nb),
        in_specs=[
            pl.BlockSpec((2, 3, _BC), lambda p, b: (0, 0, b)),
            pl.BlockSpec((NW, _BC), lambda p, b: (0, b)),
            pl.BlockSpec((4, _BC), lambda p, b: (0, b)),
            pl.BlockSpec((1, _BC), lambda p, b: (0, b)),
            pl.BlockSpec((32, 4), lambda p, b: (0, 0)),
            pl.BlockSpec((32, 1), lambda p, b: (0, 0)),
            pl.BlockSpec((16, 32), lambda p, b: (0, 0)),
            pl.BlockSpec((16, 1), lambda p, b: (0, 0)),
            pl.BlockSpec((3, 16), lambda p, b: (0, 0)),
            pl.BlockSpec((3, 1), lambda p, b: (0, 0)),
        ],
        out_specs=[
            pl.BlockSpec((3, _BC), lambda p, b: (0, b)),
            pl.BlockSpec((1, _BC), lambda p, b: (0, b)),
            pl.BlockSpec(memory_space=pltpu.SMEM),
        ],
        out_shape=[
            jax.ShapeDtypeStruct((3, NP), jnp.float32),
            jax.ShapeDtypeStruct((1, NP), jnp.int32),
            jax.ShapeDtypeStruct((1, 3), jnp.int32),
        ],
        scratch_shapes=[
            pltpu.VMEM((4, NP), jnp.float32),
            pltpu.SMEM((2,), jnp.float32),
        ],
    )(addsT, mx, tblT, rl2d, w1t, b1c, w2t, b2c, w3t, b3c)


def kernel(x, uncertainty, edge_index, edge_lengths, refinement_level,
           W1, b1, W2, b2, W3, b3):
    f32, i32 = jnp.float32, jnp.int32
    src = edge_index[0]
    dst = edge_index[1]
    pad_n = NEP - N_EDGES
    pad_idx = (N_NODES +
               (jnp.arange(pad_n, dtype=i32) % (NP - N_NODES))).astype(i32)
    src_p = jnp.concatenate([src, pad_idx])
    dst_p = jnp.concatenate([dst, pad_idx])
    len_p = jnp.concatenate([edge_lengths, jnp.ones((pad_n,), f32)])
    xt = jnp.pad(x.T, ((0, 0), (0, NP - N_NODES)))
    ut = jnp.pad(uncertainty.T, ((0, 0), (0, NP - N_NODES)))

    tblT = _pre(xt, ut)
    adds, maxp, _sn, _sv = _sc_call(tblT[0], tblT[1], tblT[2], tblT[3],
                                    src_p, dst_p, len_p)
    adds = adds.reshape(NC, 3, NP)
    mx = maxp.reshape(NW, NP)
    rl2d = jnp.pad(refinement_level, (0, NP - N_NODES),
                   constant_values=1).reshape(1, NP)
    probsT, dec, cnt = _post(adds, mx, tblT, rl2d,
                             W1.T, b1.reshape(32, 1),
                             W2.T, b2.reshape(16, 1),
                             W3.T, b3.reshape(3, 1))
    return (x, edge_index, dec[0, :N_NODES], probsT[:, :N_NODES].T,
            tblT[3, :N_NODES], cnt[0, 0], cnt[0, 1], cnt[0, 2])
